# baseline (speedup 1.0000x reference)
import jax
import jax.numpy as jnp
from jax import lax
from jax.experimental import pallas as pl
from jax.experimental.pallas import tpu as pltpu
from jax.experimental.pallas import tpu_sc as plsc

N = 10000
IN_DIM = 256
OUT_DIM = 64
NUM_HEADS = 4
ALPHA = 0.2

NT = 16
NSC = 2
CK = 128
BN = 512

NP = 10240
RPT = NP // NT
NPT = 2 * RPT


def _tc_prep(h_ref, w_ref, alr_ref, ft_ref, elr_ref, mx_ref):
    i = pl.program_id(0)
    ftb = jnp.dot(h_ref[...], w_ref[...], preferred_element_type=jnp.float32)
    ft_ref[0] = ftb[:, :128]
    ft_ref[1] = ftb[:, 128:]
    elr = jnp.dot(ftb, alr_ref[...], preferred_element_type=jnp.float32)
    elr_ref[...] = elr
    rid = i * BN + lax.broadcasted_iota(jnp.int32, (BN, 1), 0)
    m = jnp.max(jnp.where(rid < N, elr, -3.0e38), axis=0, keepdims=True)

    @pl.when(i == 0)
    def _():
        mx_ref[...] = m

    @pl.when(i > 0)
    def _():
        mx_ref[...] = jnp.maximum(mx_ref[...], m)


def _vgather(vec, idx16):
    return lax.gather(
        vec, idx16[:, None],
        dimension_numbers=lax.GatherDimensionNumbers(
            offset_dims=(), collapsed_slice_dims=(0,), start_index_map=(0,)),
        slice_sizes=(1,),
        mode=lax.GatherScatterMode.PROMISE_IN_BOUNDS)


def _sc_edges(ft_hbm, elr_hbm, cmax_hbm, srcft_hbm, dst_hbm, za_hbm, zn_hbm,
              bias_hbm, out_hbm,
              sfb0_v, sfb1_v, dsb0_v, dsb1_v, dscat0_v, dscat1_v,
              rows0_v, rows1_v, elb_v, erb_v, eidx_v, ridx_v,
              nidx0_v, nidx1_v, sfl0_v, sfl1_v, nout_v, cm_v, bias_v,
              agg_sh, normfl_sh, elrfl_sh,
              sem_lr, sem_ft0, sem_ft1, sem_n0, sem_n1, sem_a0, sem_a1,
              sem_ix0, sem_ix1):
    c = lax.axis_index("c")
    tid = lax.axis_index("s")
    ch = srcft_hbm.shape[2]
    cnp = c * NP
    sfb = (sfb0_v, sfb1_v)
    dsb = (dsb0_v, dsb1_v)
    dscat = (dscat0_v, dscat1_v)
    rows = (rows0_v, rows1_v)
    nidx = (nidx0_v, nidx1_v)
    sfl = (sfl0_v, sfl1_v)
    sem_ft = (sem_ft0, sem_ft1)
    sem_n = (sem_n0, sem_n1)
    sem_a = (sem_a0, sem_a1)
    sem_ix = (sem_ix0, sem_ix1)

    pltpu.sync_copy(cmax_hbm, cm_v)
    pltpu.sync_copy(bias_hbm, bias_v)

    pltpu.sync_copy(za_hbm, rows0_v)
    r0 = tid * RPT
    for k in range(RPT // CK):
        pltpu.sync_copy(rows0_v, agg_sh.at[pl.ds(r0 + k * CK, CK)])
    n0 = tid * NPT
    pltpu.sync_copy(zn_hbm, nout_v)
    pltpu.sync_copy(nout_v, normfl_sh.at[pl.ds(n0, NPT)])
    for b in range(2):
        sl = pl.ds(2 * n0 + b * NPT, NPT)
        pltpu.sync_copy(elr_hbm.at[c, sl], nout_v)
        pltpu.sync_copy(nout_v, elrfl_sh.at[sl])
    plsc.subcore_barrier()

    cmv = cm_v[...]
    cmb = [_vgather(cmv, jnp.full((16,), 2 * c + k, jnp.int32))
           for k in range(2)]

    def idx_load(j, b, sem):
        c0 = pltpu.async_copy(srcft_hbm.at[c, tid, j], sfb[b], sem)
        c1 = pltpu.async_copy(dst_hbm.at[tid, j], dsb[b], sem)
        return c0, c1

    def idx_wait(j, b, sem):
        pltpu.make_async_copy(srcft_hbm.at[c, tid, j], sfb[b], sem).wait()
        pltpu.make_async_copy(dst_hbm.at[tid, j], dsb[b], sem).wait()

    def ft_fire(x, sem):
        for q in range(4):
            sl = pl.ds(q * 32, 32)
            pltpu.async_copy(ft_hbm.at[sfb[x].at[sl]], rows[x].at[sl], sem)

    def ft_wait(x, sem):
        for q in range(4):
            sl = pl.ds(q * 32, 32)
            pltpu.make_async_copy(
                ft_hbm.at[sfb[x].at[sl]], rows[x].at[sl], sem).wait()

    def cycle(jj, j, b):
        for g in range(CK // 16):
            sl16 = pl.ds(g * 16, 16)
            src16 = sfb[b][sl16] - cnp
            dst16 = dsb[b][sl16]
            dscat[b][sl16] = dst16
            for k in range(2):
                osl = pl.ds(k * CK + g * 16, 16)
                eidx_v[osl] = src16 * 4 + k
                ridx_v[osl] = dst16 * 4 + (2 + k)
                nidx[b][osl] = dst16 + k * NP
        glr0 = pltpu.async_copy(elrfl_sh.at[eidx_v], elb_v, sem_lr)
        glr1 = pltpu.async_copy(elrfl_sh.at[ridx_v], erb_v, sem_lr)
        o = 1 - b
        if b == 0:
            @pl.when(jj > 0)
            def _():
                pltpu.make_async_copy(
                    sfl[o], normfl_sh.at[nidx[o]], sem_n[o]).wait()
                pltpu.make_async_copy(
                    rows[o], agg_sh.at[dscat[o]], sem_a[o]).wait()
                idx_wait(j + 1, o, sem_ix[o])
            ft_fire(o, sem_ft[o])
        else:
            pltpu.make_async_copy(
                sfl[o], normfl_sh.at[nidx[o]], sem_n[o]).wait()
            pltpu.make_async_copy(
                rows[o], agg_sh.at[dscat[o]], sem_a[o]).wait()

            @pl.when(jj < ch // 2 - 1)
            def _():
                idx_wait(j + 1, o, sem_ix[o])
                ft_fire(o, sem_ft[o])
        glr0.wait()
        glr1.wait()
        for w in range(2 * CK // 16):
            sl16 = pl.ds(w * 16, 16)
            ev = elb_v[sl16] + erb_v[sl16]
            ev = jnp.maximum(ev, ALPHA * ev)
            sfl[b][sl16] = jnp.exp(ev - cmb[w // (CK // 16)])
        pltpu.async_copy(sfl[b], normfl_sh.at[nidx[b]], sem_n[b], add=True)
        ft_wait(b, sem_ft[b])

        @pl.when(jj < ch // 2 - 1)
        def _():
            idx_load(j + 2, b, sem_ix[b])

        def gbody(g, _):
            sv0 = sfl[b][pl.ds(g * 16, 16)]
            sv1 = sfl[b][pl.ds(CK + g * 16, 16)]
            base = g * 16
            for l in range(16):
                s0 = lax.broadcast(sv0[l], (16,))
                s1 = lax.broadcast(sv1[l], (16,))
                e = base + l
                for v in range(4):
                    sl = pl.ds(v * 16, 16)
                    rows[b][e, sl] = rows[b][e, sl] * s0
                for v in range(4, 8):
                    sl = pl.ds(v * 16, 16)
                    rows[b][e, sl] = rows[b][e, sl] * s1
            return 0

        lax.fori_loop(0, CK // 16, gbody, 0)
        pltpu.async_copy(rows[b], agg_sh.at[dscat[b]], sem_a[b], add=True)

    for cp in idx_load(0, 0, sem_ix0):
        cp.wait()
    for cp in idx_load(1, 1, sem_ix1):
        cp.wait()
    ft_fire(0, sem_ft0)

    def body(jj, carry):
        cycle(jj, 2 * jj, 0)
        cycle(jj, 2 * jj + 1, 1)
        return 0

    lax.fori_loop(0, ch // 2, body, 0)
    pltpu.make_async_copy(sfl[1], normfl_sh.at[nidx[1]], sem_n[1]).wait()
    pltpu.make_async_copy(rows[1], agg_sh.at[dscat[1]], sem_a[1]).wait()
    plsc.subcore_barrier()

    pltpu.sync_copy(normfl_sh.at[pl.ds(r0, RPT)], nout_v.at[pl.ds(0, RPT)])
    pltpu.sync_copy(normfl_sh.at[pl.ds(NP + r0, RPT)],
                    nout_v.at[pl.ds(RPT, RPT)])
    bvs = [bias_v[pl.ds(c * 128 + v * 16, 16)] for v in range(8)]

    def dump_chunk(k, carry):
        sl = pl.ds(r0 + k * CK, CK)
        pltpu.sync_copy(agg_sh.at[sl], rows0_v)

        def ngroup(g, carry2):
            i0 = k * CK + g * 16
            rv0 = 1.0 / jnp.maximum(nout_v[pl.ds(i0, 16)], 1e-9)
            rv1 = 1.0 / jnp.maximum(nout_v[pl.ds(RPT + i0, 16)], 1e-9)
            for l in range(16):
                b0 = lax.broadcast(rv0[l], (16,))
                b1 = lax.broadcast(rv1[l], (16,))
                row = g * 16 + l
                for v in range(4):
                    sl16 = pl.ds(v * 16, 16)
                    rows0_v[row, sl16] = rows0_v[row, sl16] * b0 + bvs[v]
                for v in range(4, 8):
                    sl16 = pl.ds(v * 16, 16)
                    rows0_v[row, sl16] = rows0_v[row, sl16] * b1 + bvs[v]
            return 0

        lax.fori_loop(0, CK // 16, ngroup, 0)
        pltpu.sync_copy(rows0_v, out_hbm.at[sl, pl.ds(c * 128, 128)])
        return 0

    lax.fori_loop(0, RPT // CK, dump_chunk, 0)


def kernel(h, edge_index, fc_w, attn_l, attn_r, ret_bias):
    f32 = jnp.float32
    e_total = edge_index.shape[1]
    ept = -(-e_total // (NT * 2 * CK)) * 2 * CK
    e_pad = ept * NT
    ch = ept // CK

    eye4 = jnp.eye(NUM_HEADS, dtype=f32)
    al_m = (attn_l[:, :, None] * eye4[:, None, :]).reshape(IN_DIM, NUM_HEADS)
    ar_m = (attn_r[:, :, None] * eye4[:, None, :]).reshape(IN_DIM, NUM_HEADS)
    alr_m = jnp.concatenate([al_m[:, :2], ar_m[:, :2],
                             al_m[:, 2:], ar_m[:, 2:]], axis=1)

    grid = (NP // BN,)
    ft, elr_all, mx = pl.pallas_call(
        _tc_prep,
        grid=grid,
        in_specs=[
            pl.BlockSpec((BN, IN_DIM), lambda i: (i, 0)),
            pl.BlockSpec((IN_DIM, IN_DIM), lambda i: (0, 0)),
            pl.BlockSpec((IN_DIM, 8), lambda i: (0, 0)),
        ],
        out_specs=[
            pl.BlockSpec((2, BN, 128), lambda i: (0, i, 0)),
            pl.BlockSpec((BN, 8), lambda i: (i, 0)),
            pl.BlockSpec((1, 8), lambda i: (0, 0)),
        ],
        out_shape=[
            jax.ShapeDtypeStruct((2, NP, 128), f32),
            jax.ShapeDtypeStruct((NP, 8), f32),
            jax.ShapeDtypeStruct((1, 8), f32),
        ],
    )(h, fc_w, alr_m)

    cmax = jnp.maximum(mx[0, jnp.array([0, 1, 4, 5])]
                       + mx[0, jnp.array([2, 3, 6, 7])], 0.0)
    cmax16 = jnp.tile(cmax, 4)

    elr_sc = jnp.stack([
        elr_all[:, :4].reshape(-1),
        elr_all[:, 4:].reshape(-1),
    ])

    src = edge_index[0]
    dst = edge_index[1]
    npad = e_pad - e_total
    pad_dst = N + (jnp.arange(npad, dtype=jnp.int32) % (NP - N))
    src_p = jnp.concatenate([src, jnp.zeros((npad,), jnp.int32)])
    dst_p = jnp.concatenate([dst, pad_dst])
    srcft_r = jnp.stack([src_p, src_p + NP]).reshape(NSC, NT, ch, CK)
    dst_r = dst_p.reshape(NT, ch, CK)

    ft_flat = ft.reshape(NSC * NP, 128)
    za = jnp.zeros((CK, 128), f32)
    zn = jnp.zeros((NPT,), f32)

    bias_row = ret_bias.reshape(NUM_HEADS * OUT_DIM)
    mesh = plsc.VectorSubcoreMesh(core_axis_name="c", subcore_axis_name="s")
    out_pad = pl.kernel(
        _sc_edges,
        out_type=jax.ShapeDtypeStruct((NP, 256), f32),
        mesh=mesh,
        compiler_params=pltpu.CompilerParams(needs_layout_passes=False),
        scratch_types=[
            pltpu.VMEM((CK,), jnp.int32),
            pltpu.VMEM((CK,), jnp.int32),
            pltpu.VMEM((CK,), jnp.int32),
            pltpu.VMEM((CK,), jnp.int32),
            pltpu.VMEM((CK,), jnp.int32),
            pltpu.VMEM((CK,), jnp.int32),
            pltpu.VMEM((CK, 128), f32),
            pltpu.VMEM((CK, 128), f32),
            pltpu.VMEM((2 * CK,), f32),
            pltpu.VMEM((2 * CK,), f32),
            pltpu.VMEM((2 * CK,), jnp.int32),
            pltpu.VMEM((2 * CK,), jnp.int32),
            pltpu.VMEM((2 * CK,), jnp.int32),
            pltpu.VMEM((2 * CK,), jnp.int32),
            pltpu.VMEM((2 * CK,), f32),
            pltpu.VMEM((2 * CK,), f32),
            pltpu.VMEM((NPT,), f32),
            pltpu.VMEM((16,), f32),
            pltpu.VMEM((256,), f32),
            pltpu.VMEM_SHARED((NP, 128), f32),
            pltpu.VMEM_SHARED((2 * NP,), f32),
            pltpu.VMEM_SHARED((4 * NP,), f32),
            pltpu.SemaphoreType.DMA,
            pltpu.SemaphoreType.DMA,
            pltpu.SemaphoreType.DMA,
            pltpu.SemaphoreType.DMA,
            pltpu.SemaphoreType.DMA,
            pltpu.SemaphoreType.DMA,
            pltpu.SemaphoreType.DMA,
            pltpu.SemaphoreType.DMA,
            pltpu.SemaphoreType.DMA,
        ],
    )(ft_flat, elr_sc, cmax16, srcft_r, dst_r, za, zn, bias_row)

    return out_pad[:N]


# speedup vs baseline: 1.0195x; 1.0195x over previous
import jax
import jax.numpy as jnp
from jax import lax
from jax.experimental import pallas as pl
from jax.experimental.pallas import tpu as pltpu
from jax.experimental.pallas import tpu_sc as plsc

N = 10000
IN_DIM = 256
OUT_DIM = 64
NUM_HEADS = 4
ALPHA = 0.2

NT = 16
NSC = 2
CK = 128
BN = 512

NP = 10240
RPT = NP // NT
NPT = 2 * RPT


def _tc_prep(h_ref, w_ref, alr_ref, ft_ref, elr_ref, mx_ref):
    i = pl.program_id(0)
    ftb = jnp.dot(h_ref[...], w_ref[...], preferred_element_type=jnp.float32)
    ft_ref[0] = ftb[:, :128]
    ft_ref[1] = ftb[:, 128:]
    elr = jnp.dot(ftb, alr_ref[...], preferred_element_type=jnp.float32)
    elr_ref[...] = elr
    rid = i * BN + lax.broadcasted_iota(jnp.int32, (BN, 1), 0)
    m = jnp.max(jnp.where(rid < N, elr, -3.0e38), axis=0, keepdims=True)

    @pl.when(i == 0)
    def _():
        mx_ref[...] = m

    @pl.when(i > 0)
    def _():
        mx_ref[...] = jnp.maximum(mx_ref[...], m)


def _tc_finish(agg_ref, nrm_ref, bias_ref, out_ref):
    for h in range(NUM_HEADS):
        a = agg_ref[h // 2][:, (h % 2) * 64:(h % 2) * 64 + 64]
        n = jnp.maximum(nrm_ref[h // 2, h % 2][:, None], 1e-9)
        out_ref[:, h * 64:(h + 1) * 64] = a / n + bias_ref[0, h * 64:(h + 1) * 64][None, :]


def _vgather(vec, idx16):
    return lax.gather(
        vec, idx16[:, None],
        dimension_numbers=lax.GatherDimensionNumbers(
            offset_dims=(), collapsed_slice_dims=(0,), start_index_map=(0,)),
        slice_sizes=(1,),
        mode=lax.GatherScatterMode.PROMISE_IN_BOUNDS)


def _sc_edges(ft_hbm, elr_hbm, cmax_hbm, srcft_hbm, dst_hbm, za_hbm, zn_hbm,
              agg_out, norm_out,
              sfb0_v, sfb1_v, dsb0_v, dsb1_v, dscat0_v, dscat1_v,
              rows0_v, rows1_v, elb_v, erb_v, eidx_v, ridx_v,
              nidx0_v, nidx1_v, sfl0_v, sfl1_v, nout_v, cm_v,
              agg_sh, normfl_sh, elrfl_sh,
              sem_lr, sem_ft0, sem_ft1, sem_n0, sem_n1, sem_a0, sem_a1,
              sem_ix0, sem_ix1):
    c = lax.axis_index("c")
    tid = lax.axis_index("s")
    ch = srcft_hbm.shape[2]
    cnp = c * NP
    sfb = (sfb0_v, sfb1_v)
    dsb = (dsb0_v, dsb1_v)
    dscat = (dscat0_v, dscat1_v)
    rows = (rows0_v, rows1_v)
    nidx = (nidx0_v, nidx1_v)
    sfl = (sfl0_v, sfl1_v)
    sem_ft = (sem_ft0, sem_ft1)
    sem_n = (sem_n0, sem_n1)
    sem_a = (sem_a0, sem_a1)
    sem_ix = (sem_ix0, sem_ix1)

    pltpu.sync_copy(cmax_hbm, cm_v)

    pltpu.sync_copy(za_hbm, rows0_v)
    r0 = tid * RPT
    for k in range(RPT // CK):
        pltpu.sync_copy(rows0_v, agg_sh.at[pl.ds(r0 + k * CK, CK)])
    n0 = tid * NPT
    pltpu.sync_copy(zn_hbm, nout_v)
    pltpu.sync_copy(nout_v, normfl_sh.at[pl.ds(n0, NPT)])
    for b in range(2):
        sl = pl.ds(2 * n0 + b * NPT, NPT)
        pltpu.sync_copy(elr_hbm.at[c, sl], nout_v)
        pltpu.sync_copy(nout_v, elrfl_sh.at[sl])
    plsc.subcore_barrier()

    cmv = cm_v[...]
    cmb = [_vgather(cmv, jnp.full((16,), 2 * c + k, jnp.int32))
           for k in range(2)]

    def idx_load(j, b, sem):
        c0 = pltpu.async_copy(srcft_hbm.at[c, tid, j], sfb[b], sem)
        c1 = pltpu.async_copy(dst_hbm.at[tid, j], dsb[b], sem)
        return c0, c1

    def idx_wait(j, b, sem):
        pltpu.make_async_copy(srcft_hbm.at[c, tid, j], sfb[b], sem).wait()
        pltpu.make_async_copy(dst_hbm.at[tid, j], dsb[b], sem).wait()

    def ft_fire(x, sem):
        for q in range(4):
            sl = pl.ds(q * 32, 32)
            pltpu.async_copy(ft_hbm.at[sfb[x].at[sl]], rows[x].at[sl], sem)

    def ft_wait(x, sem):
        for q in range(4):
            sl = pl.ds(q * 32, 32)
            pltpu.make_async_copy(
                ft_hbm.at[sfb[x].at[sl]], rows[x].at[sl], sem).wait()

    def build_fire(slot):
        for g in range(CK // 16):
            sl16 = pl.ds(g * 16, 16)
            src16 = sfb[slot][sl16] - cnp
            dst16 = dsb[slot][sl16]
            dscat[slot][sl16] = dst16
            for k in range(2):
                osl = pl.ds(k * CK + g * 16, 16)
                eidx_v[osl] = src16 * 4 + k
                ridx_v[osl] = dst16 * 4 + (2 + k)
                nidx[slot][osl] = dst16 + k * NP
        pltpu.async_copy(elrfl_sh.at[eidx_v], elb_v, sem_lr)
        pltpu.async_copy(elrfl_sh.at[ridx_v], erb_v, sem_lr)

    def cycle(jj, j, b):
        o = 1 - b
        if b == 0:
            @pl.when(jj > 0)
            def _():
                pltpu.make_async_copy(
                    sfl[o], normfl_sh.at[nidx[o]], sem_n[o]).wait()
                pltpu.make_async_copy(
                    rows[o], agg_sh.at[dscat[o]], sem_a[o]).wait()
                idx_wait(j + 1, o, sem_ix[o])
            ft_fire(o, sem_ft[o])
        else:
            pltpu.make_async_copy(
                sfl[o], normfl_sh.at[nidx[o]], sem_n[o]).wait()
            pltpu.make_async_copy(
                rows[o], agg_sh.at[dscat[o]], sem_a[o]).wait()

            @pl.when(jj < ch // 2 - 1)
            def _():
                idx_wait(j + 1, o, sem_ix[o])
                ft_fire(o, sem_ft[o])
        pltpu.make_async_copy(elrfl_sh.at[eidx_v], elb_v, sem_lr).wait()
        pltpu.make_async_copy(elrfl_sh.at[ridx_v], erb_v, sem_lr).wait()
        for w in range(2 * CK // 16):
            sl16 = pl.ds(w * 16, 16)
            ev = elb_v[sl16] + erb_v[sl16]
            ev = jnp.maximum(ev, ALPHA * ev)
            sfl[b][sl16] = jnp.exp(ev - cmb[w // (CK // 16)])
        pltpu.async_copy(sfl[b], normfl_sh.at[nidx[b]], sem_n[b], add=True)
        if b == 0:
            build_fire(o)
        else:
            @pl.when(jj < ch // 2 - 1)
            def _():
                build_fire(o)
        ft_wait(b, sem_ft[b])

        @pl.when(jj < ch // 2 - 1)
        def _():
            idx_load(j + 2, b, sem_ix[b])

        def gbody(g, _):
            sv0 = sfl[b][pl.ds(g * 16, 16)]
            sv1 = sfl[b][pl.ds(CK + g * 16, 16)]
            base = g * 16
            for l in range(16):
                s0 = lax.broadcast(sv0[l], (16,))
                s1 = lax.broadcast(sv1[l], (16,))
                e = base + l
                for v in range(4):
                    sl = pl.ds(v * 16, 16)
                    rows[b][e, sl] = rows[b][e, sl] * s0
                for v in range(4, 8):
                    sl = pl.ds(v * 16, 16)
                    rows[b][e, sl] = rows[b][e, sl] * s1
            return 0

        lax.fori_loop(0, CK // 16, gbody, 0)
        pltpu.async_copy(rows[b], agg_sh.at[dscat[b]], sem_a[b], add=True)

    for cp in idx_load(0, 0, sem_ix0):
        cp.wait()
    for cp in idx_load(1, 1, sem_ix1):
        cp.wait()
    build_fire(0)
    ft_fire(0, sem_ft0)

    def body(jj, carry):
        cycle(jj, 2 * jj, 0)
        cycle(jj, 2 * jj + 1, 1)
        return 0

    lax.fori_loop(0, ch // 2, body, 0)
    pltpu.make_async_copy(sfl[1], normfl_sh.at[nidx[1]], sem_n[1]).wait()
    pltpu.make_async_copy(rows[1], agg_sh.at[dscat[1]], sem_a[1]).wait()
    plsc.subcore_barrier()

    for k in range(RPT // CK):
        sl = pl.ds(r0 + k * CK, CK)
        pltpu.sync_copy(agg_sh.at[sl], rows0_v)
        pltpu.sync_copy(rows0_v, agg_out.at[c, sl])
    nsl = pl.ds(n0, NPT)
    pltpu.sync_copy(normfl_sh.at[nsl], nout_v)
    pltpu.sync_copy(nout_v, norm_out.at[c, nsl])


def kernel(h, edge_index, fc_w, attn_l, attn_r, ret_bias):
    f32 = jnp.float32
    e_total = edge_index.shape[1]
    ept = -(-e_total // (NT * 2 * CK)) * 2 * CK
    e_pad = ept * NT
    ch = ept // CK

    eye4 = jnp.eye(NUM_HEADS, dtype=f32)
    al_m = (attn_l[:, :, None] * eye4[:, None, :]).reshape(IN_DIM, NUM_HEADS)
    ar_m = (attn_r[:, :, None] * eye4[:, None, :]).reshape(IN_DIM, NUM_HEADS)
    alr_m = jnp.concatenate([al_m[:, :2], ar_m[:, :2],
                             al_m[:, 2:], ar_m[:, 2:]], axis=1)

    grid = (NP // BN,)
    ft, elr_all, mx = pl.pallas_call(
        _tc_prep,
        grid=grid,
        in_specs=[
            pl.BlockSpec((BN, IN_DIM), lambda i: (i, 0)),
            pl.BlockSpec((IN_DIM, IN_DIM), lambda i: (0, 0)),
            pl.BlockSpec((IN_DIM, 8), lambda i: (0, 0)),
        ],
        out_specs=[
            pl.BlockSpec((2, BN, 128), lambda i: (0, i, 0)),
            pl.BlockSpec((BN, 8), lambda i: (i, 0)),
            pl.BlockSpec((1, 8), lambda i: (0, 0)),
        ],
        out_shape=[
            jax.ShapeDtypeStruct((2, NP, 128), f32),
            jax.ShapeDtypeStruct((NP, 8), f32),
            jax.ShapeDtypeStruct((1, 8), f32),
        ],
    )(h, fc_w, alr_m)

    cmax = jnp.maximum(mx[0, jnp.array([0, 1, 4, 5])]
                       + mx[0, jnp.array([2, 3, 6, 7])], 0.0)
    cmax16 = jnp.tile(cmax, 4)

    elr_sc = jnp.stack([
        elr_all[:, :4].reshape(-1),
        elr_all[:, 4:].reshape(-1),
    ])

    src = edge_index[0]
    dst = edge_index[1]
    npad = e_pad - e_total
    pad_dst = N + (jnp.arange(npad, dtype=jnp.int32) % (NP - N))
    src_p = jnp.concatenate([src, jnp.zeros((npad,), jnp.int32)])
    dst_p = jnp.concatenate([dst, pad_dst])
    srcft_r = jnp.stack([src_p, src_p + NP]).reshape(NSC, NT, ch, CK)
    dst_r = dst_p.reshape(NT, ch, CK)

    ft_flat = ft.reshape(NSC * NP, 128)
    za = jnp.zeros((CK, 128), f32)
    zn = jnp.zeros((NPT,), f32)

    mesh = plsc.VectorSubcoreMesh(core_axis_name="c", subcore_axis_name="s")
    agg, normfl = pl.kernel(
        _sc_edges,
        out_type=[
            jax.ShapeDtypeStruct((NSC, NP, 128), f32),
            jax.ShapeDtypeStruct((NSC, 2 * NP), f32),
        ],
        mesh=mesh,
        compiler_params=pltpu.CompilerParams(needs_layout_passes=False),
        scratch_types=[
            pltpu.VMEM((CK,), jnp.int32),
            pltpu.VMEM((CK,), jnp.int32),
            pltpu.VMEM((CK,), jnp.int32),
            pltpu.VMEM((CK,), jnp.int32),
            pltpu.VMEM((CK,), jnp.int32),
            pltpu.VMEM((CK,), jnp.int32),
            pltpu.VMEM((CK, 128), f32),
            pltpu.VMEM((CK, 128), f32),
            pltpu.VMEM((2 * CK,), f32),
            pltpu.VMEM((2 * CK,), f32),
            pltpu.VMEM((2 * CK,), jnp.int32),
            pltpu.VMEM((2 * CK,), jnp.int32),
            pltpu.VMEM((2 * CK,), jnp.int32),
            pltpu.VMEM((2 * CK,), jnp.int32),
            pltpu.VMEM((2 * CK,), f32),
            pltpu.VMEM((2 * CK,), f32),
            pltpu.VMEM((NPT,), f32),
            pltpu.VMEM((16,), f32),
            pltpu.VMEM_SHARED((NP, 128), f32),
            pltpu.VMEM_SHARED((2 * NP,), f32),
            pltpu.VMEM_SHARED((4 * NP,), f32),
            pltpu.SemaphoreType.DMA,
            pltpu.SemaphoreType.DMA,
            pltpu.SemaphoreType.DMA,
            pltpu.SemaphoreType.DMA,
            pltpu.SemaphoreType.DMA,
            pltpu.SemaphoreType.DMA,
            pltpu.SemaphoreType.DMA,
            pltpu.SemaphoreType.DMA,
            pltpu.SemaphoreType.DMA,
        ],
    )(ft_flat, elr_sc, cmax16, srcft_r, dst_r, za, zn)

    bias_row = ret_bias.reshape(1, NUM_HEADS * OUT_DIM)
    norm2 = normfl.reshape(NSC, 2, NP)
    out = pl.pallas_call(
        _tc_finish,
        grid=grid,
        in_specs=[
            pl.BlockSpec((2, BN, 128), lambda i: (0, i, 0)),
            pl.BlockSpec((2, 2, BN), lambda i: (0, 0, i)),
            pl.BlockSpec((1, 256), lambda i: (0, 0)),
        ],
        out_specs=pl.BlockSpec((BN, 256), lambda i: (i, 0)),
        out_shape=jax.ShapeDtypeStruct((N, 256), f32),
    )(agg, norm2, bias_row)

    return out
